# Initial kernel scaffold; baseline (speedup 1.0000x reference)
import functools

import jax
import jax.numpy as jnp
from jax import lax
from jax.experimental import pallas as pl
from jax.experimental.pallas import tpu as pltpu
from jax.experimental.pallas import tpu_sc as plsc

N_NODES = 10000
N_EDGES = 320000
D = 128

NC = 2
NS = 16
NW = NC * NS
EPW = N_EDGES // NW
C = 400
NCHUNK = EPW // C

_mesh = plsc.VectorSubcoreMesh(
    core_axis_name="c", subcore_axis_name="s", num_cores=NC, num_subcores=NS
)


def _rsqrt_nr(s):
    y = plsc.bitcast(jnp.int32(0x5F3759DF) - (plsc.bitcast(s, jnp.int32) >> 1),
                     jnp.float32)
    h = 0.5 * s
    y = y * (1.5 - h * y * y)
    y = y * (1.5 - h * y * y)
    y = y * (1.5 - h * y * y)
    return y


@functools.partial(
    pl.kernel,
    out_type=jax.ShapeDtypeStruct((N_EDGES,), jnp.float32),
    mesh=_mesh,
    scratch_types=[
        pltpu.VMEM((C,), jnp.int32),
        pltpu.VMEM((C,), jnp.int32),
        pltpu.VMEM((C, D), jnp.float32),
        pltpu.VMEM((C, D), jnp.float32),
        pltpu.VMEM((C,), jnp.float32),
        pltpu.VMEM((C,), jnp.float32),
        pltpu.SemaphoreType.DMA,
        pltpu.SemaphoreType.DMA,
    ],
)
def _score_kernel(head_hbm, tail_hbm, x_hbm, sw_hbm, out_hbm,
                  hidx_v, tidx_v, hrows_v, trows_v, sw_v, out_v,
                  sem_h, sem_t):
    wid = lax.axis_index("s") * NC + lax.axis_index("c")
    base = wid * EPW

    def chunk_body(g, carry):
        off = base + g * C
        pltpu.sync_copy(head_hbm.at[pl.ds(off, C)], hidx_v)
        pltpu.sync_copy(tail_hbm.at[pl.ds(off, C)], tidx_v)
        pltpu.sync_copy(sw_hbm.at[pl.ds(off, C)], sw_v)
        cph = pltpu.async_copy(x_hbm.at[hidx_v], hrows_v, sem_h)
        cpt = pltpu.async_copy(x_hbm.at[tidx_v], trows_v, sem_t)
        cph.wait()
        cpt.wait()

        def edge_body(e, c2):
            acc = jnp.zeros((16,), jnp.float32)
            for j in range(D // 16):
                h = hrows_v[e, pl.ds(j * 16, 16)]
                t = trows_v[e, pl.ds(j * 16, 16)]
                d = (h - t) + 1e-6
                acc = acc + d * d
            out_v[e] = jnp.sum(acc)
            return c2

        lax.fori_loop(0, C, edge_body, 0)

        def grp_body(k, c3):
            ss = out_v[pl.ds(k * 16, 16)]
            y = _rsqrt_nr(jnp.maximum(ss, 1e-12))
            out_v[pl.ds(k * 16, 16)] = ss * y * sw_v[pl.ds(k * 16, 16)]
            return c3

        lax.fori_loop(0, C // 16, grp_body, 0)
        pltpu.sync_copy(out_v, out_hbm.at[pl.ds(off, C)])
        return carry

    lax.fori_loop(0, NCHUNK, chunk_body, 0)


def kernel(tuples, x, sw):
    head = tuples[:, 0]
    tail = tuples[:, 1]
    return _score_kernel(head, tail, x, sw)


# speedup vs baseline: 15.1892x; 15.1892x over previous
import functools

import jax
import jax.numpy as jnp
from jax import lax
from jax.experimental import pallas as pl
from jax.experimental.pallas import tpu as pltpu
from jax.experimental.pallas import tpu_sc as plsc

N_NODES = 10000
N_EDGES = 320000
D = 128

NC = 2
NS = 16
NW = NC * NS
EPW = N_EDGES // NW
C = 400
NCHUNK = EPW // C

_mesh = plsc.VectorSubcoreMesh(
    core_axis_name="c", subcore_axis_name="s", num_cores=NC, num_subcores=NS
)


def _rsqrt_nr(s):
    y = plsc.bitcast(jnp.int32(0x5F3759DF) - (plsc.bitcast(s, jnp.int32) >> 1),
                     jnp.float32)
    h = 0.5 * s
    y = y * (1.5 - h * y * y)
    y = y * (1.5 - h * y * y)
    y = y * (1.5 - h * y * y)
    return y


@functools.partial(
    pl.kernel,
    out_type=jax.ShapeDtypeStruct((N_EDGES,), jnp.float32),
    mesh=_mesh,
    compiler_params=pltpu.CompilerParams(needs_layout_passes=False),
    scratch_types=[
        pltpu.VMEM((C,), jnp.int32),
        pltpu.VMEM((C,), jnp.int32),
        pltpu.VMEM((C, D), jnp.float32),
        pltpu.VMEM((C, D), jnp.float32),
        pltpu.VMEM((C,), jnp.float32),
        pltpu.VMEM((C,), jnp.float32),
        pltpu.SemaphoreType.DMA,
        pltpu.SemaphoreType.DMA,
    ],
)
def _score_kernel(head_hbm, tail_hbm, x_hbm, sw_hbm, out_hbm,
                  hidx_v, tidx_v, hrows_v, trows_v, sw_v, out_v,
                  sem_h, sem_t):
    wid = lax.axis_index("s") * NC + lax.axis_index("c")
    base = wid * EPW

    def chunk_body(g, carry):
        off = base + g * C
        pltpu.sync_copy(head_hbm.at[pl.ds(off, C)], hidx_v)
        pltpu.sync_copy(tail_hbm.at[pl.ds(off, C)], tidx_v)
        pltpu.sync_copy(sw_hbm.at[pl.ds(off, C)], sw_v)
        cph = pltpu.async_copy(x_hbm.at[hidx_v], hrows_v, sem_h)
        cpt = pltpu.async_copy(x_hbm.at[tidx_v], trows_v, sem_t)
        cph.wait()
        cpt.wait()

        lane = lax.iota(jnp.int32, 16)

        def grp_body(k, c2):
            rbase = k * 16
            ssvec = jnp.zeros((16,), jnp.float32)
            for i in range(16):
                e = rbase + i
                acc = jnp.zeros((16,), jnp.float32)
                for j in range(D // 16):
                    h = hrows_v[e, pl.ds(j * 16, 16)]
                    t = trows_v[e, pl.ds(j * 16, 16)]
                    d = (h - t) + 1e-6
                    acc = acc + d * d
                ssvec = jnp.where(lane == i, jnp.sum(acc), ssvec)
            y = _rsqrt_nr(jnp.maximum(ssvec, 1e-12))
            out_v[pl.ds(rbase, 16)] = ssvec * y * sw_v[pl.ds(rbase, 16)]
            return c2

        lax.fori_loop(0, C // 16, grp_body, 0)
        pltpu.sync_copy(out_v, out_hbm.at[pl.ds(off, C)])
        return carry

    lax.fori_loop(0, NCHUNK, chunk_body, 0)


def kernel(tuples, x, sw):
    head = tuples[:, 0]
    tail = tuples[:, 1]
    return _score_kernel(head, tail, x, sw)


# speedup vs baseline: 19.6334x; 1.2926x over previous
import functools

import jax
import jax.numpy as jnp
from jax import lax
from jax.experimental import pallas as pl
from jax.experimental.pallas import tpu as pltpu
from jax.experimental.pallas import tpu_sc as plsc

N_NODES = 10000
N_EDGES = 320000
D = 128

NC = 2
NS = 16
NW = NC * NS
EPW = N_EDGES // NW
C = 80
NCHUNK = EPW // C

_mesh = plsc.VectorSubcoreMesh(
    core_axis_name="c", subcore_axis_name="s", num_cores=NC, num_subcores=NS
)


def _rsqrt_nr(s):
    y = plsc.bitcast(jnp.int32(0x5F3759DF) - (plsc.bitcast(s, jnp.int32) >> 1),
                     jnp.float32)
    h = 0.5 * s
    y = y * (1.5 - h * y * y)
    y = y * (1.5 - h * y * y)
    y = y * (1.5 - h * y * y)
    return y


@functools.partial(
    pl.kernel,
    out_type=jax.ShapeDtypeStruct((N_EDGES,), jnp.float32),
    mesh=_mesh,
    compiler_params=pltpu.CompilerParams(needs_layout_passes=False),
    scratch_types=[
        pltpu.VMEM((EPW,), jnp.int32),
        pltpu.VMEM((EPW,), jnp.int32),
        pltpu.VMEM((EPW,), jnp.float32),
        pltpu.VMEM((EPW,), jnp.float32),
        pltpu.VMEM((C, D), jnp.float32),
        pltpu.VMEM((C, D), jnp.float32),
        pltpu.VMEM((C, D), jnp.float32),
        pltpu.VMEM((C, D), jnp.float32),
        pltpu.SemaphoreType.DMA,
        pltpu.SemaphoreType.DMA,
    ],
)
def _score_kernel(head_hbm, tail_hbm, x_hbm, sw_hbm, out_hbm,
                  hidx_v, tidx_v, sw_v, score_v,
                  hr_a, tr_a, hr_b, tr_b, sem_a, sem_b):
    wid = lax.axis_index("s") * NC + lax.axis_index("c")
    base = pl.multiple_of(wid * EPW, EPW)

    pltpu.sync_copy(head_hbm.at[pl.ds(base, EPW)], hidx_v)
    pltpu.sync_copy(tail_hbm.at[pl.ds(base, EPW)], tidx_v)
    pltpu.sync_copy(sw_hbm.at[pl.ds(base, EPW)], sw_v)

    lane = lax.iota(jnp.int32, 16)

    def mk_gathers(g, hr, tr, sem):
        off = pl.multiple_of(g * C, C)
        ch = pltpu.make_async_copy(x_hbm.at[hidx_v.at[pl.ds(off, C)]], hr, sem)
        ct = pltpu.make_async_copy(x_hbm.at[tidx_v.at[pl.ds(off, C)]], tr, sem)
        return ch, ct

    def start(g, hr, tr, sem):
        ch, ct = mk_gathers(g, hr, tr, sem)
        ch.start()
        ct.start()

    def wait(g, hr, tr, sem):
        ch, ct = mk_gathers(g, hr, tr, sem)
        ch.wait()
        ct.wait()

    def compute(g, hr, tr):
        cbase = pl.multiple_of(g * C, C)

        def grp_body(kk, c2):
            rbase = kk * 16
            ssvec = jnp.zeros((16,), jnp.float32)
            for i in range(16):
                acc = jnp.zeros((16,), jnp.float32)
                for j in range(D // 16):
                    h = hr[rbase + i, pl.ds(j * 16, 16)]
                    t = tr[rbase + i, pl.ds(j * 16, 16)]
                    d = (h - t) + 1e-6
                    acc = acc + d * d
                ssvec = jnp.where(lane == i, jnp.sum(acc), ssvec)
            y = _rsqrt_nr(jnp.maximum(ssvec, 1e-12))
            sl = pl.ds(cbase + rbase, 16)
            score_v[sl] = ssvec * y * sw_v[sl]
            return c2

        lax.fori_loop(0, C // 16, grp_body, 0)

    start(0, hr_a, tr_a, sem_a)
    start(1, hr_b, tr_b, sem_b)

    def pair_body(k, carry):
        g = 2 * k
        wait(g, hr_a, tr_a, sem_a)
        compute(g, hr_a, tr_a)
        start(g + 2, hr_a, tr_a, sem_a)
        wait(g + 1, hr_b, tr_b, sem_b)
        compute(g + 1, hr_b, tr_b)

        @pl.when(k < NCHUNK // 2 - 1)
        def _():
            start(g + 3, hr_b, tr_b, sem_b)

        return carry

    lax.fori_loop(0, NCHUNK // 2, pair_body, 0)
    wait(NCHUNK - 1, hr_a, tr_a, sem_a)
    compute(NCHUNK - 1, hr_a, tr_a)

    pltpu.sync_copy(score_v, out_hbm.at[pl.ds(base, EPW)])


def kernel(tuples, x, sw):
    head = tuples[:, 0]
    tail = tuples[:, 1]
    return _score_kernel(head, tail, x, sw)
